# baseline (speedup 1.0000x reference)
import functools

import jax
import jax.numpy as jnp
from jax import lax
from jax.experimental import pallas as pl
from jax.experimental.pallas import tpu as pltpu
from jax.experimental.pallas import tpu_sc as plsc

EMBED_DIM = 32
LANES = 16
NUM_CORES = 2
NUM_SUBCORES = 16
NUM_WORKERS = NUM_CORES * NUM_SUBCORES


def _make_kernel(batch):
    b_per_w = batch // NUM_WORKERS
    n_groups = b_per_w // LANES
    mesh = plsc.VectorSubcoreMesh(
        core_axis_name="c", subcore_axis_name="s", num_cores=NUM_CORES
    )

    @functools.partial(
        pl.kernel,
        out_type=jax.ShapeDtypeStruct((batch,), jnp.float32),
        mesh=mesh,
        scratch_types=[
            pltpu.VMEM((b_per_w,), jnp.int32),
            pltpu.VMEM((b_per_w,), jnp.int32),
            pltpu.VMEM((b_per_w, EMBED_DIM), jnp.float32),
            pltpu.VMEM((b_per_w, EMBED_DIM), jnp.float32),
            pltpu.VMEM((b_per_w,), jnp.float32),
            pltpu.VMEM((b_per_w,), jnp.float32),
            pltpu.VMEM((b_per_w,), jnp.float32),
            pltpu.SemaphoreType.DMA,
        ],
        compiler_params=pltpu.CompilerParams(
            needs_layout_passes=False, use_tc_tiling_on_sc=False),
    )
    def two_tower(users_hbm, items_hbm, uemb_hbm, iemb_hbm, ub_hbm, ib_hbm,
                  out_hbm, uidx_v, iidx_v, urows_v, irows_v, ubias_v, ibias_v,
                  out_v, sem):
        wid = lax.axis_index("s") * NUM_CORES + lax.axis_index("c")
        base = wid * b_per_w

        pltpu.sync_copy(users_hbm.at[pl.ds(base, b_per_w)], uidx_v)
        pltpu.sync_copy(items_hbm.at[pl.ds(base, b_per_w)], iidx_v)

        cp_u = pltpu.async_copy(uemb_hbm.at[uidx_v], urows_v, sem)
        cp_i = pltpu.async_copy(iemb_hbm.at[iidx_v], irows_v, sem)
        cp_ub = pltpu.async_copy(ub_hbm.at[uidx_v], ubias_v, sem)
        cp_ib = pltpu.async_copy(ib_hbm.at[iidx_v], ibias_v, sem)
        cp_u.wait()
        cp_i.wait()
        cp_ub.wait()
        cp_ib.wait()

        def group(g, carry):
            off = pl.multiple_of(g * LANES, LANES)
            rows = lax.iota(jnp.int32, LANES) + g * LANES
            acc = ubias_v[pl.ds(off, LANES)] + ibias_v[pl.ds(off, LANES)]
            for d in range(EMBED_DIM):
                dcol = jnp.full((LANES,), d, jnp.int32)
                ucol = plsc.load_gather(urows_v, [rows, dcol])
                icol = plsc.load_gather(irows_v, [rows, dcol])
                acc = acc + ucol * icol
            out_v[pl.ds(off, LANES)] = acc
            return carry

        lax.fori_loop(0, n_groups, group, 0)
        pltpu.sync_copy(out_v, out_hbm.at[pl.ds(base, b_per_w)])

    return two_tower


def kernel(users, items, user_emb, item_emb, ub, ib):
    batch = users.shape[0]
    fn = _make_kernel(batch)
    return fn(users, items, user_emb, item_emb,
              ub.reshape(-1), ib.reshape(-1))
